# baseline (speedup 1.0000x reference)
import jax
import jax.numpy as jnp
from jax.experimental import pallas as pl
from jax.experimental.pallas import tpu as pltpu

ALPHA_ = 0.1
BM = 256


def _prop_kernel(adj_ref, x_ref, h_ref, o_ref):
    i = pl.program_id(0)
    o_ref[...] = (1.0 - ALPHA_) * jnp.dot(
        adj_ref[...], x_ref[...], preferred_element_type=jnp.float32
    ) + ALPHA_ * h_ref[pl.ds(i * BM, BM), :]


@jax.jit
def kernel(x, adj, h):
    n, d = x.shape
    nm = n // BM
    return pl.pallas_call(
        _prop_kernel,
        grid=(nm,),
        in_specs=[
            pl.BlockSpec((BM, n), lambda i: (i, 0)),
            pl.BlockSpec((n, d), lambda i: (0, 0)),
            pl.BlockSpec((n, d), lambda i: (0, 0)),
        ],
        out_specs=pl.BlockSpec((BM, d), lambda i: (i, 0)),
        out_shape=jax.ShapeDtypeStruct((n, d), jnp.float32),
        compiler_params=pltpu.CompilerParams(
            dimension_semantics=("parallel",),
        ),
    )(adj, x, h)


# speedup vs baseline: 1.0676x; 1.0676x over previous
import jax
import jax.numpy as jnp
from jax.experimental import pallas as pl
from jax.experimental.pallas import tpu as pltpu

ALPHA_ = 0.1
BM = 512


def _prop_kernel(adj_top_ref, adj_bot_ref, x_ref, h_ref, o_ref):
    i = pl.program_id(0)
    half = h_ref.shape[0] // 2
    xv = x_ref[...]
    sl = pl.ds(i * BM, BM)
    o_ref[0] = (1.0 - ALPHA_) * jnp.dot(
        adj_top_ref[...], xv, preferred_element_type=jnp.float32
    ) + ALPHA_ * h_ref[sl, :]
    sl2 = pl.ds(half + i * BM, BM)
    o_ref[1] = (1.0 - ALPHA_) * jnp.dot(
        adj_bot_ref[...], xv, preferred_element_type=jnp.float32
    ) + ALPHA_ * h_ref[sl2, :]


@jax.jit
def kernel(x, adj, h):
    n, d = x.shape
    half = n // 2
    nm = half // BM
    out = pl.pallas_call(
        _prop_kernel,
        grid=(nm,),
        in_specs=[
            pl.BlockSpec((BM, n), lambda i: (i, 0)),
            pl.BlockSpec((BM, n), lambda i, _nm=nm: (i + _nm, 0)),
            pl.BlockSpec((n, d), lambda i: (0, 0)),
            pl.BlockSpec((n, d), lambda i: (0, 0)),
        ],
        out_specs=pl.BlockSpec((2, BM, d), lambda i: (0, i, 0)),
        out_shape=jax.ShapeDtypeStruct((2, half, d), jnp.float32),
        compiler_params=pltpu.CompilerParams(
            dimension_semantics=("parallel",),
        ),
    )(adj, adj, x, h)
    return out.reshape(n, d)


# speedup vs baseline: 1.1378x; 1.0657x over previous
import jax
import jax.numpy as jnp
from jax.experimental import pallas as pl
from jax.experimental.pallas import tpu as pltpu

ALPHA_ = 0.1
BM = 512


def _prop_kernel(adj_ref, x_ref, h_ref, o_ref):
    i = pl.program_id(0)
    sl = pl.ds(i * BM, BM)
    o_ref[sl, :] = (1.0 - ALPHA_) * jnp.dot(
        adj_ref[...], x_ref[...], preferred_element_type=jnp.float32
    ) + ALPHA_ * h_ref[sl, :]


@jax.jit
def kernel(x, adj, h):
    n, d = x.shape
    nm = n // BM
    return pl.pallas_call(
        _prop_kernel,
        grid=(nm,),
        in_specs=[
            pl.BlockSpec((BM, n), lambda i: (i, 0)),
            pl.BlockSpec((n, d), lambda i: (0, 0)),
            pl.BlockSpec((n, d), lambda i: (0, 0)),
        ],
        out_specs=pl.BlockSpec((n, d), lambda i: (0, 0)),
        out_shape=jax.ShapeDtypeStruct((n, d), jnp.float32),
        compiler_params=pltpu.CompilerParams(
            dimension_semantics=("parallel",),
        ),
    )(adj, x, h)
